# Initial kernel scaffold; baseline (speedup 1.0000x reference)
import jax
import jax.numpy as jnp
from jax.experimental import pallas as pl


def kernel(x, edge_index, W1, b1, W2, b2, Wlin, blin):
    raise NotImplementedError("write your pallas kernel here")


# speedup vs baseline: 19.5744x; 19.5744x over previous
import functools

import jax
import jax.numpy as jnp
from jax import lax
from jax.experimental import pallas as pl
from jax.experimental.pallas import tpu as pltpu
from jax.experimental.pallas import tpu_sc as plsc

N = 10000
D = 128
E = 320000

NC = 2
NS = 16
NW = NC * NS

CH = 128
STEPS = 79
EPW = STEPS * CH
EP = NW * EPW
NP = 10112
RPT = NP // NS
NBLK = NP // 128
NB = 2

_mesh = plsc.VectorSubcoreMesh(core_axis_name="c", subcore_axis_name="s")


@functools.partial(
    pl.kernel,
    out_type=jax.ShapeDtypeStruct((NC * NP,), jnp.float32),
    mesh=_mesh,
    scratch_types=[
        pltpu.VMEM((STEPS, CH), jnp.int32),
        pltpu.VMEM((CH,), jnp.float32),
        pltpu.VMEM((RPT,), jnp.float32),
        pltpu.VMEM_SHARED((NP,), jnp.float32),
    ],
)
def _sc_degree(dst_hbm, out_hbm, didx_all, ones, zbuf, acc):
    c = lax.axis_index("c")
    s = lax.axis_index("s")
    wid = c * NS + s
    one16 = jnp.full((16,), 1.0, jnp.float32)
    zero16 = jnp.zeros((16,), jnp.float32)
    for j in range(CH // 16):
        ones[pl.ds(j * 16, 16)] = one16

    def _zfill(k, carry):
        zbuf[pl.ds(k * 16, 16)] = zero16
        return carry

    lax.fori_loop(0, RPT // 16, _zfill, 0)
    pltpu.sync_copy(zbuf, acc.at[pl.ds(s * RPT, RPT)])
    plsc.subcore_barrier()

    pltpu.sync_copy(dst_hbm.at[wid], didx_all)

    def _step(t, carry):
        pltpu.sync_copy(ones, acc.at[didx_all.at[t]], add=True)
        return carry

    lax.fori_loop(0, STEPS, _step, 0)
    plsc.subcore_barrier()
    pltpu.sync_copy(acc.at[pl.ds(s * RPT, RPT)], zbuf)
    pltpu.sync_copy(zbuf, out_hbm.at[pl.ds(c * NP + s * RPT, RPT)])


@functools.partial(
    pl.kernel,
    out_type=jax.ShapeDtypeStruct((NC * NP, D), jnp.float32),
    mesh=_mesh,
    scratch_types=[
        [pltpu.VMEM((CH,), jnp.int32) for _ in range(NB)],
        [pltpu.VMEM((CH,), jnp.int32) for _ in range(NB)],
        [pltpu.VMEM((CH, D), jnp.float32) for _ in range(NB)],
        [pltpu.SemaphoreType.DMA for _ in range(NB)],
        pltpu.VMEM_SHARED((NP, D), jnp.float32),
    ],
)
def _sc_rowscat(tab_hbm, src_hbm, dst_hbm, out_hbm,
                sidx, didx, rows, sems, acc):
    c = lax.axis_index("c")
    s = lax.axis_index("s")
    wid = c * NS + s
    zero16 = jnp.zeros((16,), jnp.float32)

    def _zfill(k, carry):
        rows[0][k // 8, pl.ds((k % 8) * 16, 16)] = zero16
        return carry

    lax.fori_loop(0, CH * 8, _zfill, 0)
    for i in range(5):
        blk = s + i * NS

        @pl.when(blk < NBLK)
        def _zcopy():
            pltpu.sync_copy(rows[0], acc.at[pl.ds(blk * CH, CH)])

    plsc.subcore_barrier()

    for b in range(NB):
        pltpu.sync_copy(src_hbm.at[wid, b], sidx[b])
        pltpu.sync_copy(dst_hbm.at[wid, b], didx[b])
        pltpu.async_copy(tab_hbm.at[sidx[b]], rows[b], sems[b])

    def _group(g, carry):
        for b in range(NB):
            t = g * NB + b

            @pl.when(t < STEPS)
            def _drain():
                pltpu.make_async_copy(
                    tab_hbm.at[sidx[b]], rows[b], sems[b]).wait()
                pltpu.sync_copy(rows[b], acc.at[didx[b]], add=True)

            @pl.when(t + NB < STEPS)
            def _fire():
                pltpu.sync_copy(src_hbm.at[wid, t + NB], sidx[b])
                pltpu.sync_copy(dst_hbm.at[wid, t + NB], didx[b])
                pltpu.async_copy(tab_hbm.at[sidx[b]], rows[b], sems[b])
        return carry

    lax.fori_loop(0, (STEPS + NB - 1) // NB, _group, 0)
    plsc.subcore_barrier()
    for i in range(5):
        blk = s + i * NS

        @pl.when(blk < NBLK)
        def _wout():
            pltpu.sync_copy(acc.at[pl.ds(blk * CH, CH)], rows[0])
            pltpu.sync_copy(rows[0], out_hbm.at[pl.ds(c * NP + blk * CH, CH)])


def _tc1_body(xp_ref, w1_ref, d0_ref, d1_ref, hs_ref, dinv_ref):
    deg = d0_ref[...] + d1_ref[...] + 1.0
    dinv = lax.rsqrt(deg)
    h = jnp.dot(xp_ref[...], w1_ref[...], preferred_element_type=jnp.float32)
    hs_ref[...] = h * dinv
    dinv_ref[...] = dinv


def _tc2_body(p0_ref, p1_ref, hs_ref, dinv_ref, b1_ref, w2_ref, out_ref):
    dinv = dinv_ref[...]
    z = dinv * (p0_ref[...] + p1_ref[...] + hs_ref[...]) + b1_ref[...]
    z = jnp.maximum(z, 0.0)
    h2 = jnp.dot(z, w2_ref[...], preferred_element_type=jnp.float32)
    out_ref[...] = h2 * dinv


def _tc3_body(q0_ref, q1_ref, hs_ref, dinv_ref, b2_ref, wd_ref, bd_ref,
              out_ref):
    dinv = dinv_ref[...]
    out2 = dinv * (q0_ref[...] + q1_ref[...] + hs_ref[...]) + b2_ref[...]
    d = jnp.sum(out2 * wd_ref[...], axis=1, keepdims=True) + bd_ref[0, 0]
    p0 = jax.nn.sigmoid(d)
    p1 = jax.nn.sigmoid(-d)
    out_ref[...] = jnp.concatenate([p0, p1], axis=1)


_blk = lambda i: (i, 0)
_rep = lambda i: (0, 0)


def _row_spec(w):
    return pl.BlockSpec((128, w), _blk)


def _full_spec(h, w):
    return pl.BlockSpec((h, w), _rep)


_tc1 = pl.pallas_call(
    _tc1_body,
    grid=(NBLK,),
    in_specs=[_row_spec(D), _full_spec(D, D), _row_spec(1), _row_spec(1)],
    out_specs=[_row_spec(D), _row_spec(1)],
    out_shape=[jax.ShapeDtypeStruct((NP, D), jnp.float32),
               jax.ShapeDtypeStruct((NP, 1), jnp.float32)],
)

_tc2 = pl.pallas_call(
    _tc2_body,
    grid=(NBLK,),
    in_specs=[_row_spec(D), _row_spec(D), _row_spec(D), _row_spec(1),
              _full_spec(1, D), _full_spec(D, D)],
    out_specs=_row_spec(D),
    out_shape=jax.ShapeDtypeStruct((NP, D), jnp.float32),
)

_tc3 = pl.pallas_call(
    _tc3_body,
    grid=(NBLK,),
    in_specs=[_row_spec(D), _row_spec(D), _row_spec(D), _row_spec(1),
              _full_spec(1, D), _full_spec(1, D), _full_spec(1, 1)],
    out_specs=pl.BlockSpec((128, 2), _blk),
    out_shape=jax.ShapeDtypeStruct((NP, 2), jnp.float32),
)


def kernel(x, edge_index, W1, b1, W2, b2, Wlin, blin):
    src = edge_index[0].astype(jnp.int32)
    dst = edge_index[1].astype(jnp.int32)
    npad = EP - E
    pad_idx = N + (jnp.arange(npad, dtype=jnp.int32) % (NP - N))
    srcp = jnp.concatenate([src, pad_idx]).reshape(NW, STEPS, CH)
    dstp = jnp.concatenate([dst, pad_idx]).reshape(NW, STEPS, CH)
    xp = jnp.pad(x, ((0, NP - N), (0, 0)))

    degp = _sc_degree(dstp)
    deg0 = degp[:NP].reshape(NP, 1)
    deg1 = degp[NP:].reshape(NP, 1)

    hs1, dinv = _tc1(xp, W1, deg0, deg1)

    p = _sc_rowscat(hs1, srcp, dstp)
    hs2 = _tc2(p[:NP], p[NP:], hs1, dinv, b1.reshape(1, D), W2)

    q = _sc_rowscat(hs2, srcp, dstp)
    wd = (Wlin[:, 0] - Wlin[:, 1]).reshape(1, D)
    bd = (blin[0] - blin[1]).reshape(1, 1)
    preds = _tc3(q[:NP], q[NP:], hs2, dinv, b2.reshape(1, D), wd, bd)
    return preds[:N]


# speedup vs baseline: 22.7748x; 1.1635x over previous
import functools

import jax
import jax.numpy as jnp
from jax import lax
from jax.experimental import pallas as pl
from jax.experimental.pallas import tpu as pltpu
from jax.experimental.pallas import tpu_sc as plsc

N = 10000
D = 128
E = 320000

NC = 2
NS = 16
NW = NC * NS

CH = 128
K = 8
NG = 10
STEPS = NG * K
EPW = STEPS * CH
EP = NW * EPW
NP = 10112
RPT = NP // NS
NBLK = NP // 128
NB = 2

_mesh = plsc.VectorSubcoreMesh(core_axis_name="c", subcore_axis_name="s")


@functools.partial(
    pl.kernel,
    out_type=jax.ShapeDtypeStruct((NC * NP,), jnp.float32),
    mesh=_mesh,
    scratch_types=[
        pltpu.VMEM((STEPS, CH), jnp.int32),
        pltpu.VMEM((CH,), jnp.float32),
        pltpu.VMEM((RPT,), jnp.float32),
        pltpu.VMEM_SHARED((NP,), jnp.float32),
    ],
)
def _sc_degree(dst_hbm, out_hbm, didx_all, ones, zbuf, acc):
    c = lax.axis_index("c")
    s = lax.axis_index("s")
    wid = c * NS + s
    one16 = jnp.full((16,), 1.0, jnp.float32)
    zero16 = jnp.zeros((16,), jnp.float32)
    for j in range(CH // 16):
        ones[pl.ds(j * 16, 16)] = one16

    def _zfill(k, carry):
        zbuf[pl.ds(k * 16, 16)] = zero16
        return carry

    lax.fori_loop(0, RPT // 16, _zfill, 0)
    pltpu.sync_copy(zbuf, acc.at[pl.ds(s * RPT, RPT)])
    plsc.subcore_barrier()

    pltpu.sync_copy(dst_hbm.at[wid], didx_all)

    def _step(t, carry):
        pltpu.sync_copy(ones, acc.at[didx_all.at[t]], add=True)
        return carry

    lax.fori_loop(0, STEPS, _step, 0)
    plsc.subcore_barrier()
    pltpu.sync_copy(acc.at[pl.ds(s * RPT, RPT)], zbuf)
    pltpu.sync_copy(zbuf, out_hbm.at[pl.ds(c * NP + s * RPT, RPT)])


@functools.partial(
    pl.kernel,
    out_type=jax.ShapeDtypeStruct((NC * NP, D), jnp.float32),
    mesh=_mesh,
    scratch_types=[
        [pltpu.VMEM((K, 2, CH), jnp.int32) for _ in range(2)],
        [pltpu.VMEM((CH, D), jnp.float32) for _ in range(NB)],
        [pltpu.SemaphoreType.DMA for _ in range(NB)],
        [pltpu.SemaphoreType.DMA for _ in range(NB)],
        pltpu.VMEM_SHARED((NP, D), jnp.float32),
    ],
)
def _sc_rowscat(tab_hbm, idx_hbm, out_hbm, ibuf, rows, gsems, ssems, acc):
    c = lax.axis_index("c")
    s = lax.axis_index("s")
    wid = c * NS + s
    zero16 = jnp.zeros((16,), jnp.float32)

    def _zfill(k, carry):
        rows[0][k // 8, pl.ds((k % 8) * 16, 16)] = zero16
        return carry

    lax.fori_loop(0, CH * 8, _zfill, 0)
    for i in range(5):
        blk = s + i * NS

        @pl.when(blk < NBLK)
        def _zcopy():
            pltpu.sync_copy(rows[0], acc.at[pl.ds(blk * CH, CH)])

    plsc.subcore_barrier()

    def _gather(t, b):
        g2, k2 = divmod(t, K)
        pltpu.async_copy(tab_hbm.at[ibuf[g2 % 2].at[k2, 0]], rows[b],
                         gsems[b])

    def _gwait(t, b):
        g2, k2 = divmod(t, K)
        pltpu.make_async_copy(tab_hbm.at[ibuf[g2 % 2].at[k2, 0]], rows[b],
                              gsems[b]).wait()

    def _scatter(t, b):
        g2, k2 = divmod(t, K)
        pltpu.async_copy(rows[b], acc.at[ibuf[g2 % 2].at[k2, 1]], ssems[b],
                         add=True)

    def _swait(t, b):
        g2, k2 = divmod(t, K)
        pltpu.make_async_copy(rows[b], acc.at[ibuf[g2 % 2].at[k2, 1]],
                              ssems[b]).wait()

    pltpu.sync_copy(idx_hbm.at[wid, 0], ibuf[0])
    for b in range(NB):
        _gather(b, b)
    for g in range(NG):
        for k in range(K):
            t = g * K + k
            b = t % NB
            _gwait(t, b)
            _scatter(t, b)
            if t + NB < STEPS:
                _swait(t, b)
                _gather(t + NB, b)

            if k == NB and g + 1 < NG:
                pltpu.sync_copy(idx_hbm.at[wid, g + 1], ibuf[(g + 1) % 2])
    for t in (STEPS - NB, STEPS - 1):
        _swait(t, t % NB)
    plsc.subcore_barrier()
    for i in range(5):
        blk = s + i * NS

        @pl.when(blk < NBLK)
        def _wout():
            pltpu.sync_copy(acc.at[pl.ds(blk * CH, CH)], rows[0])
            pltpu.sync_copy(rows[0], out_hbm.at[pl.ds(c * NP + blk * CH, CH)])


def _tc1_body(xp_ref, w1_ref, d0_ref, d1_ref, hs_ref, dinv_ref):
    deg = d0_ref[...] + d1_ref[...] + 1.0
    dinv = lax.rsqrt(deg)
    h = jnp.dot(xp_ref[...], w1_ref[...], preferred_element_type=jnp.float32)
    hs_ref[...] = h * dinv
    dinv_ref[...] = dinv


def _tc2_body(p0_ref, p1_ref, hs_ref, dinv_ref, b1_ref, w2_ref, out_ref):
    dinv = dinv_ref[...]
    z = dinv * (p0_ref[...] + p1_ref[...] + hs_ref[...]) + b1_ref[...]
    z = jnp.maximum(z, 0.0)
    h2 = jnp.dot(z, w2_ref[...], preferred_element_type=jnp.float32)
    out_ref[...] = h2 * dinv


def _tc3_body(q0_ref, q1_ref, hs_ref, dinv_ref, b2_ref, wd_ref, bd_ref,
              out_ref):
    dinv = dinv_ref[...]
    out2 = dinv * (q0_ref[...] + q1_ref[...] + hs_ref[...]) + b2_ref[...]
    d = jnp.sum(out2 * wd_ref[...], axis=1, keepdims=True) + bd_ref[0, 0]
    p0 = jax.nn.sigmoid(d)
    p1 = jax.nn.sigmoid(-d)
    out_ref[...] = jnp.concatenate([p0, p1], axis=1)


_blk = lambda i: (i, 0)
_rep = lambda i: (0, 0)


def _row_spec(w):
    return pl.BlockSpec((128, w), _blk)


def _full_spec(h, w):
    return pl.BlockSpec((h, w), _rep)


_tc1 = pl.pallas_call(
    _tc1_body,
    grid=(NBLK,),
    in_specs=[_row_spec(D), _full_spec(D, D), _row_spec(1), _row_spec(1)],
    out_specs=[_row_spec(D), _row_spec(1)],
    out_shape=[jax.ShapeDtypeStruct((NP, D), jnp.float32),
               jax.ShapeDtypeStruct((NP, 1), jnp.float32)],
)

_tc2 = pl.pallas_call(
    _tc2_body,
    grid=(NBLK,),
    in_specs=[_row_spec(D), _row_spec(D), _row_spec(D), _row_spec(1),
              _full_spec(1, D), _full_spec(D, D)],
    out_specs=_row_spec(D),
    out_shape=jax.ShapeDtypeStruct((NP, D), jnp.float32),
)

_tc3 = pl.pallas_call(
    _tc3_body,
    grid=(NBLK,),
    in_specs=[_row_spec(D), _row_spec(D), _row_spec(D), _row_spec(1),
              _full_spec(1, D), _full_spec(1, D), _full_spec(1, 1)],
    out_specs=pl.BlockSpec((128, 2), _blk),
    out_shape=jax.ShapeDtypeStruct((NP, 2), jnp.float32),
)


def kernel(x, edge_index, W1, b1, W2, b2, Wlin, blin):
    src = edge_index[0].astype(jnp.int32)
    dst = edge_index[1].astype(jnp.int32)
    npad = EP - E
    pad_idx = N + (jnp.arange(npad, dtype=jnp.int32) % (NP - N))
    srcp = jnp.concatenate([src, pad_idx])
    dstp = jnp.concatenate([dst, pad_idx])
    idxp = jnp.stack([srcp.reshape(NW, NG, K, CH),
                      dstp.reshape(NW, NG, K, CH)], axis=3)
    xp = jnp.pad(x, ((0, NP - N), (0, 0)))

    degp = _sc_degree(dstp.reshape(NW, STEPS, CH))
    deg0 = degp[:NP].reshape(NP, 1)
    deg1 = degp[NP:].reshape(NP, 1)

    hs1, dinv = _tc1(xp, W1, deg0, deg1)

    p = _sc_rowscat(hs1, idxp)
    hs2 = _tc2(p[:NP], p[NP:], hs1, dinv, b1.reshape(1, D), W2)

    q = _sc_rowscat(hs2, idxp)
    wd = (Wlin[:, 0] - Wlin[:, 1]).reshape(1, D)
    bd = (blin[0] - blin[1]).reshape(1, 1)
    preds = _tc3(q[:NP], q[NP:], hs2, dinv, b2.reshape(1, D), wd, bd)
    return preds[:N]


# speedup vs baseline: 23.1885x; 1.0182x over previous
import functools

import jax
import jax.numpy as jnp
from jax import lax
from jax.experimental import pallas as pl
from jax.experimental.pallas import tpu as pltpu
from jax.experimental.pallas import tpu_sc as plsc

N = 10000
D = 128
E = 320000

NC = 2
NS = 16
NW = NC * NS

CH = 128
K = 8
NG = 10
STEPS = NG * K
EPW = STEPS * CH
EP = NW * EPW
NP = 10112
RPT = NP // NS
NBLK = NP // 128
NB = 2

_mesh = plsc.VectorSubcoreMesh(core_axis_name="c", subcore_axis_name="s")


@functools.partial(
    pl.kernel,
    out_type=jax.ShapeDtypeStruct((NC * NP,), jnp.float32),
    mesh=_mesh,
    scratch_types=[
        pltpu.VMEM((STEPS, CH), jnp.int32),
        pltpu.VMEM((CH,), jnp.float32),
        pltpu.VMEM((RPT,), jnp.float32),
        pltpu.VMEM_SHARED((NP,), jnp.float32),
    ],
)
def _sc_degree(dst_hbm, out_hbm, didx_all, ones, zbuf, acc):
    c = lax.axis_index("c")
    s = lax.axis_index("s")
    wid = c * NS + s
    one16 = jnp.full((16,), 1.0, jnp.float32)
    zero16 = jnp.zeros((16,), jnp.float32)
    for j in range(CH // 16):
        ones[pl.ds(j * 16, 16)] = one16

    def _zfill(k, carry):
        zbuf[pl.ds(k * 16, 16)] = zero16
        return carry

    lax.fori_loop(0, RPT // 16, _zfill, 0)
    pltpu.sync_copy(zbuf, acc.at[pl.ds(s * RPT, RPT)])
    plsc.subcore_barrier()

    pltpu.sync_copy(dst_hbm.at[wid], didx_all)

    def _step(t, carry):
        pltpu.sync_copy(ones, acc.at[didx_all.at[t]], add=True)
        return carry

    lax.fori_loop(0, STEPS, _step, 0)
    plsc.subcore_barrier()
    pltpu.sync_copy(acc.at[pl.ds(s * RPT, RPT)], zbuf)
    pltpu.sync_copy(zbuf, out_hbm.at[pl.ds(c * NP + s * RPT, RPT)])


@functools.partial(
    pl.kernel,
    out_type=jax.ShapeDtypeStruct((NC * NP, D), jnp.float32),
    mesh=_mesh,
    scratch_types=[
        [pltpu.VMEM((K, 2, CH), jnp.int32) for _ in range(2)],
        [pltpu.VMEM((CH, D), jnp.float32) for _ in range(NB)],
        [pltpu.SemaphoreType.DMA for _ in range(NB)],
        [pltpu.SemaphoreType.DMA for _ in range(NB)],
        pltpu.VMEM_SHARED((NP, D), jnp.float32),
    ],
)
def _sc_rowscat(tab_hbm, idx_hbm, out_hbm, ibuf, rows, gsems, ssems, acc):
    c = lax.axis_index("c")
    s = lax.axis_index("s")
    wid = c * NS + s
    zero16 = jnp.zeros((16,), jnp.float32)

    def _zfill(k, carry):
        rows[0][k // 8, pl.ds((k % 8) * 16, 16)] = zero16
        return carry

    lax.fori_loop(0, CH * 8, _zfill, 0)
    for i in range(5):
        blk = s + i * NS

        @pl.when(blk < NBLK)
        def _zcopy():
            pltpu.sync_copy(rows[0], acc.at[pl.ds(blk * CH, CH)])

    plsc.subcore_barrier()

    def _gather(t, b):
        g2, k2 = divmod(t, K)
        pltpu.async_copy(tab_hbm.at[ibuf[g2 % 2].at[k2, 0]], rows[b],
                         gsems[b])

    def _gwait(t, b):
        g2, k2 = divmod(t, K)
        pltpu.make_async_copy(tab_hbm.at[ibuf[g2 % 2].at[k2, 0]], rows[b],
                              gsems[b]).wait()

    def _scatter(t, b):
        g2, k2 = divmod(t, K)
        pltpu.async_copy(rows[b], acc.at[ibuf[g2 % 2].at[k2, 1]], ssems[b],
                         add=True)

    def _swait(t, b):
        g2, k2 = divmod(t, K)
        pltpu.make_async_copy(rows[b], acc.at[ibuf[g2 % 2].at[k2, 1]],
                              ssems[b]).wait()

    pltpu.sync_copy(idx_hbm.at[wid, 0], ibuf[0])
    for b in range(NB):
        _gather(b, b)
    for g in range(NG):
        for k in range(K):
            t = g * K + k
            b = t % NB
            _gwait(t, b)
            _scatter(t, b)
            if t + NB < STEPS:
                _swait(t, b)
                _gather(t + NB, b)

            if k == NB and g + 1 < NG:
                pltpu.sync_copy(idx_hbm.at[wid, g + 1], ibuf[(g + 1) % 2])
    for t in (STEPS - NB, STEPS - 1):
        _swait(t, t % NB)
    plsc.subcore_barrier()
    for i in range(5):
        blk = s + i * NS

        @pl.when(blk < NBLK)
        def _wout():
            pltpu.sync_copy(acc.at[pl.ds(blk * CH, CH)], rows[0])
            pltpu.sync_copy(rows[0], out_hbm.at[pl.ds(c * NP + blk * CH, CH)])


def _col_rep(row):
    return jnp.broadcast_to(jnp.transpose(row), (128, 128))


def _tc1_body(xp_ref, w1_ref, d0_ref, d1_ref, hs_ref, dinv_ref):
    deg = d0_ref[0] + d1_ref[0] + 1.0
    dinv = lax.rsqrt(deg)
    dinv_ref[0] = dinv
    h = jnp.dot(xp_ref[...], w1_ref[...], preferred_element_type=jnp.float32)
    hs_ref[...] = h * _col_rep(dinv)


def _tc2_body(p0_ref, p1_ref, hs_ref, dinv_ref, b1_ref, w2_ref, out_ref):
    rep = _col_rep(dinv_ref[0])
    z = rep * (p0_ref[...] + p1_ref[...] + hs_ref[...]) + b1_ref[...]
    z = jnp.maximum(z, 0.0)
    h2 = jnp.dot(z, w2_ref[...], preferred_element_type=jnp.float32)
    out_ref[...] = h2 * rep


def _tc3_body(q0_ref, q1_ref, hs_ref, dinv_ref, b2_ref, wd_ref, bd_ref,
              out_ref):
    rep = _col_rep(dinv_ref[0])
    out2 = rep * (q0_ref[...] + q1_ref[...] + hs_ref[...]) + b2_ref[...]
    d = jnp.sum(out2 * wd_ref[...], axis=1, keepdims=True) + bd_ref[0, 0]
    p0 = jax.nn.sigmoid(d)
    p1 = jax.nn.sigmoid(-d)
    out_ref[...] = jnp.concatenate([p0, p1], axis=1)


_blk = lambda i: (i, 0)
_rep = lambda i: (0, 0)


def _row_spec(w):
    return pl.BlockSpec((128, w), _blk)


def _full_spec(h, w):
    return pl.BlockSpec((h, w), _rep)


_blk3 = lambda i: (i, 0, 0)
_blk3b = lambda i: (i + NBLK, 0, 0)
_lane_spec = pl.BlockSpec((1, 1, D), _blk3)

_tc1 = pl.pallas_call(
    _tc1_body,
    grid=(NBLK,),
    in_specs=[_row_spec(D), _full_spec(D, D), _lane_spec,
              pl.BlockSpec((1, 1, D), _blk3b)],
    out_specs=[_row_spec(D), _lane_spec],
    out_shape=[jax.ShapeDtypeStruct((NP, D), jnp.float32),
               jax.ShapeDtypeStruct((NBLK, 1, D), jnp.float32)],
)

_tc2 = pl.pallas_call(
    _tc2_body,
    grid=(NBLK,),
    in_specs=[_row_spec(D), _row_spec(D), _row_spec(D), _lane_spec,
              _full_spec(1, D), _full_spec(D, D)],
    out_specs=_row_spec(D),
    out_shape=jax.ShapeDtypeStruct((NP, D), jnp.float32),
)

_tc3 = pl.pallas_call(
    _tc3_body,
    grid=(NBLK,),
    in_specs=[_row_spec(D), _row_spec(D), _row_spec(D), _lane_spec,
              _full_spec(1, D), _full_spec(1, D), _full_spec(1, 1)],
    out_specs=pl.BlockSpec((128, 2), _blk),
    out_shape=jax.ShapeDtypeStruct((NP, 2), jnp.float32),
)


def kernel(x, edge_index, W1, b1, W2, b2, Wlin, blin):
    src = edge_index[0].astype(jnp.int32)
    dst = edge_index[1].astype(jnp.int32)
    npad = EP - E
    pad_idx = N + (jnp.arange(npad, dtype=jnp.int32) % (NP - N))
    srcp = jnp.concatenate([src, pad_idx])
    dstp = jnp.concatenate([dst, pad_idx])
    idxp = jnp.stack([srcp.reshape(NW, NG, K, CH),
                      dstp.reshape(NW, NG, K, CH)], axis=3)
    xp = jnp.pad(x, ((0, NP - N), (0, 0)))

    degp = _sc_degree(dstp.reshape(NW, STEPS, CH))
    deg3d = degp.reshape(2 * NBLK, 1, D)

    hs1, dinv = _tc1(xp, W1, deg3d, deg3d)

    p = _sc_rowscat(hs1, idxp)
    hs2 = _tc2(p[:NP], p[NP:], hs1, dinv, b1.reshape(1, D), W2)

    q = _sc_rowscat(hs2, idxp)
    wd = (Wlin[:, 0] - Wlin[:, 1]).reshape(1, D)
    bd = (blin[0] - blin[1]).reshape(1, 1)
    preds = _tc3(q[:NP], q[NP:], hs2, dinv, b2.reshape(1, D), wd, bd)
    return preds[:N]


# speedup vs baseline: 31.3688x; 1.3528x over previous
import functools

import jax
import jax.numpy as jnp
from jax import lax
from jax.experimental import pallas as pl
from jax.experimental.pallas import tpu as pltpu
from jax.experimental.pallas import tpu_sc as plsc

N = 10000
D = 128
E = 320000

NC = 2
NS = 16
NW = NC * NS

CH = 128
K = 8
NG = 10
STEPS = NG * K
EPW = STEPS * CH
EP = NW * EPW
NP = 10240
RPT = NP // NS
NBLK = NP // 128
GR = 1024
GB = NP // GR
NB = 2

_mesh = plsc.VectorSubcoreMesh(core_axis_name="c", subcore_axis_name="s")


@functools.partial(
    pl.kernel,
    out_type=jax.ShapeDtypeStruct((NC * NP,), jnp.float32),
    mesh=_mesh,
    scratch_types=[
        pltpu.VMEM((STEPS, CH), jnp.int32),
        pltpu.VMEM((CH,), jnp.float32),
        pltpu.VMEM((RPT,), jnp.float32),
        pltpu.VMEM_SHARED((NP,), jnp.float32),
    ],
)
def _sc_degree(dst_hbm, out_hbm, didx_all, ones, zbuf, acc):
    c = lax.axis_index("c")
    s = lax.axis_index("s")
    wid = c * NS + s
    one16 = jnp.full((16,), 1.0, jnp.float32)
    zero16 = jnp.zeros((16,), jnp.float32)
    for j in range(CH // 16):
        ones[pl.ds(j * 16, 16)] = one16

    def _zfill(k, carry):
        zbuf[pl.ds(k * 16, 16)] = zero16
        return carry

    lax.fori_loop(0, RPT // 16, _zfill, 0)
    pltpu.sync_copy(zbuf, acc.at[pl.ds(s * RPT, RPT)])
    plsc.subcore_barrier()

    pltpu.sync_copy(dst_hbm.at[wid], didx_all)

    def _step(t, carry):
        pltpu.sync_copy(ones, acc.at[didx_all.at[t]], add=True)
        return carry

    lax.fori_loop(0, STEPS, _step, 0)
    plsc.subcore_barrier()
    pltpu.sync_copy(acc.at[pl.ds(s * RPT, RPT)], zbuf)
    pltpu.sync_copy(zbuf, out_hbm.at[pl.ds(c * NP + s * RPT, RPT)])


@functools.partial(
    pl.kernel,
    out_type=jax.ShapeDtypeStruct((NC * NP, D), jnp.float32),
    mesh=_mesh,
    scratch_types=[
        [pltpu.VMEM((K, 2, CH), jnp.int32) for _ in range(2)],
        [pltpu.VMEM((CH, D), jnp.float32) for _ in range(NB)],
        [pltpu.SemaphoreType.DMA for _ in range(NB)],
        [pltpu.SemaphoreType.DMA for _ in range(NB)],
        pltpu.VMEM_SHARED((NP, D), jnp.float32),
    ],
)
def _sc_rowscat(tab_hbm, idx_hbm, out_hbm, ibuf, rows, gsems, ssems, acc):
    c = lax.axis_index("c")
    s = lax.axis_index("s")
    wid = c * NS + s
    zero16 = jnp.zeros((16,), jnp.float32)

    def _zfill(k, carry):
        rows[0][k // 8, pl.ds((k % 8) * 16, 16)] = zero16
        return carry

    lax.fori_loop(0, CH * 8, _zfill, 0)
    for i in range(NBLK // NS):
        blk = s + i * NS
        pltpu.sync_copy(rows[0], acc.at[pl.ds(blk * CH, CH)])
    plsc.subcore_barrier()

    def _gather(t, b):
        g2, k2 = divmod(t, K)
        pltpu.async_copy(tab_hbm.at[ibuf[g2 % 2].at[k2, 0]], rows[b],
                         gsems[b])

    def _gwait(t, b):
        g2, k2 = divmod(t, K)
        pltpu.make_async_copy(tab_hbm.at[ibuf[g2 % 2].at[k2, 0]], rows[b],
                              gsems[b]).wait()

    def _scatter(t, b):
        g2, k2 = divmod(t, K)
        pltpu.async_copy(rows[b], acc.at[ibuf[g2 % 2].at[k2, 1]], ssems[b],
                         add=True)

    def _swait(t, b):
        g2, k2 = divmod(t, K)
        pltpu.make_async_copy(rows[b], acc.at[ibuf[g2 % 2].at[k2, 1]],
                              ssems[b]).wait()

    pltpu.sync_copy(idx_hbm.at[wid, 0], ibuf[0])
    for b in range(NB):
        _gather(b, b)
    for g in range(NG):
        for k in range(K):
            t = g * K + k
            b = t % NB
            _gwait(t, b)
            _scatter(t, b)
            if t + NB < STEPS:
                _swait(t, b)
                _gather(t + NB, b)

            if k == NB and g + 1 < NG:
                pltpu.sync_copy(idx_hbm.at[wid, g + 1], ibuf[(g + 1) % 2])
    for t in (STEPS - NB, STEPS - 1):
        _swait(t, t % NB)
    plsc.subcore_barrier()
    for i in range(NBLK // NS):
        blk = s + i * NS
        pltpu.sync_copy(acc.at[pl.ds(blk * CH, CH)], rows[0])
        pltpu.sync_copy(rows[0], out_hbm.at[pl.ds(c * NP + blk * CH, CH)])


def _rep_block(d):
    return jnp.concatenate(
        [jnp.broadcast_to(jnp.transpose(d[j]), (128, 128))
         for j in range(GR // 128)], axis=0)


def _tc1_body(xp_ref, w1_ref, d0_ref, d1_ref, hs_ref, dinv_ref):
    deg = d0_ref[...] + d1_ref[...] + 1.0
    dinv = lax.rsqrt(deg)
    dinv_ref[...] = dinv
    h = jnp.dot(xp_ref[...], w1_ref[...], preferred_element_type=jnp.float32)
    hs_ref[...] = h * _rep_block(dinv)


def _tc2_body(p0_ref, p1_ref, hs_ref, dinv_ref, b1_ref, w2_ref, out_ref):
    rep = _rep_block(dinv_ref[...])
    z = rep * (p0_ref[...] + p1_ref[...] + hs_ref[...]) + b1_ref[...]
    z = jnp.maximum(z, 0.0)
    h2 = jnp.dot(z, w2_ref[...], preferred_element_type=jnp.float32)
    out_ref[...] = h2 * rep


def _tc3_body(q0_ref, q1_ref, hs_ref, dinv_ref, b2_ref, wd_ref, bd_ref,
              out_ref):
    rep = _rep_block(dinv_ref[...])
    out2 = rep * (q0_ref[...] + q1_ref[...] + hs_ref[...]) + b2_ref[...]
    d = jnp.sum(out2 * wd_ref[...], axis=1, keepdims=True) + bd_ref[0, 0]
    p0 = jax.nn.sigmoid(d)
    p1 = jax.nn.sigmoid(-d)
    out_ref[...] = jnp.concatenate([p0, p1], axis=1)


_blk = lambda i: (i, 0)
_rep0 = lambda i: (0, 0)
_blk3 = lambda i: (i, 0, 0)
_blk3b = lambda i: (i + NBLK // (GR // 128), 0, 0)

_row_spec = pl.BlockSpec((GR, D), _blk)
_lane_spec = pl.BlockSpec((GR // 128, 1, D), _blk3)


def _full_spec(h, w):
    return pl.BlockSpec((h, w), _rep0)


_tc1 = pl.pallas_call(
    _tc1_body,
    grid=(GB,),
    in_specs=[_row_spec, _full_spec(D, D), _lane_spec,
              pl.BlockSpec((GR // 128, 1, D), _blk3b)],
    out_specs=[_row_spec, _lane_spec],
    out_shape=[jax.ShapeDtypeStruct((NP, D), jnp.float32),
               jax.ShapeDtypeStruct((NBLK, 1, D), jnp.float32)],
)

_tc2 = pl.pallas_call(
    _tc2_body,
    grid=(GB,),
    in_specs=[_row_spec, _row_spec, _row_spec, _lane_spec,
              _full_spec(1, D), _full_spec(D, D)],
    out_specs=_row_spec,
    out_shape=jax.ShapeDtypeStruct((NP, D), jnp.float32),
)

_tc3 = pl.pallas_call(
    _tc3_body,
    grid=(GB,),
    in_specs=[_row_spec, _row_spec, _row_spec, _lane_spec,
              _full_spec(1, D), _full_spec(1, D), _full_spec(1, 1)],
    out_specs=pl.BlockSpec((GR, 2), _blk),
    out_shape=jax.ShapeDtypeStruct((NP, 2), jnp.float32),
)


def kernel(x, edge_index, W1, b1, W2, b2, Wlin, blin):
    src = edge_index[0].astype(jnp.int32)
    dst = edge_index[1].astype(jnp.int32)
    npad = EP - E
    pad_idx = N + (jnp.arange(npad, dtype=jnp.int32) % (NP - N))
    srcp = jnp.concatenate([src, pad_idx])
    dstp = jnp.concatenate([dst, pad_idx])
    idxp = jnp.stack([srcp.reshape(NW, NG, K, CH),
                      dstp.reshape(NW, NG, K, CH)], axis=3)
    xp = jnp.pad(x, ((0, NP - N), (0, 0)))

    degp = _sc_degree(dstp.reshape(NW, STEPS, CH))
    deg3d = degp.reshape(2 * NBLK, 1, D)

    hs1, dinv = _tc1(xp, W1, deg3d, deg3d)

    p = _sc_rowscat(hs1, idxp)
    hs2 = _tc2(p[:NP], p[NP:], hs1, dinv, b1.reshape(1, D), W2)

    q = _sc_rowscat(hs2, idxp)
    wd = (Wlin[:, 0] - Wlin[:, 1]).reshape(1, D)
    bd = (blin[0] - blin[1]).reshape(1, 1)
    preds = _tc3(q[:NP], q[NP:], hs2, dinv, b2.reshape(1, D), wd, bd)
    return preds[:N]
